# baseline (speedup 1.0000x reference)
import functools

import jax
import jax.numpy as jnp
import numpy as np
from jax import lax
from jax.experimental import pallas as pl
from jax.experimental.pallas import tpu as pltpu
from jax.experimental.pallas import tpu_sc as plsc

N_TOKENS = 1000000
EMBED_DIM = 64
BATCH = 4096
SEQ = 200

_NC = 2
_NS = 16
_NW = _NC * _NS

_SEQ_PER_CHUNK = 2
_CHUNK = _SEQ_PER_CHUNK * SEQ
_PER_W = (BATCH // _NW) * SEQ
_NCHUNK = _PER_W // _CHUNK
_LANES = 16
_DBLK = EMBED_DIM // _LANES


def _make_pos_enc():
    pos = np.arange(SEQ, dtype=np.float32)[:, None]
    i = np.arange(0, EMBED_DIM, 2, dtype=np.float32)[None, :]
    angle = pos / np.power(10000.0, i / float(EMBED_DIM))
    pe = np.zeros((SEQ, EMBED_DIM), dtype=np.float32)
    pe[:, 0::2] = np.sin(angle)
    pe[:, 1::2] = np.cos(angle)
    return jnp.asarray(pe)


def _sc_kernel(table_hbm, idx_hbm, pe_hbm, out_hbm, idx_v, rows_v, pe_v, sem):
    wid = lax.axis_index("s") * _NC + lax.axis_index("c")
    base = wid * _PER_W
    pltpu.sync_copy(pe_hbm, pe_v)

    def chunk_body(c, carry):
        off = base + c * _CHUNK
        pltpu.sync_copy(idx_hbm.at[pl.ds(off, _CHUNK)], idx_v)
        pltpu.async_copy(table_hbm.at[idx_v], rows_v, sem).wait()

        def add_body(s, carry2):
            srow = s % SEQ
            for d in range(_DBLK):
                rows_v[s, pl.ds(d * _LANES, _LANES)] = (
                    rows_v[s, pl.ds(d * _LANES, _LANES)]
                    + pe_v[srow, pl.ds(d * _LANES, _LANES)]
                )
            return carry2

        lax.fori_loop(0, _CHUNK, add_body, 0, unroll=2)
        pltpu.sync_copy(rows_v, out_hbm.at[pl.ds(off, _CHUNK)])
        return carry

    lax.fori_loop(0, _NCHUNK, chunk_body, 0)


@jax.jit
def _gather_pe(inputs, table, pe):
    idx = inputs.reshape(-1)
    mesh = plsc.VectorSubcoreMesh(core_axis_name="c", subcore_axis_name="s")
    k = functools.partial(
        pl.kernel,
        mesh=mesh,
        out_type=jax.ShapeDtypeStruct((BATCH * SEQ, EMBED_DIM), jnp.float32),
        scratch_types=[
            pltpu.VMEM((_CHUNK,), jnp.int32),
            pltpu.VMEM((_CHUNK, EMBED_DIM), jnp.float32),
            pltpu.VMEM((SEQ, EMBED_DIM), jnp.float32),
            pltpu.SemaphoreType.DMA,
        ],
        compiler_params=pltpu.CompilerParams(use_tc_tiling_on_sc=False),
    )(_sc_kernel)
    out = k(table, idx, pe)
    return out.reshape(BATCH, SEQ, EMBED_DIM)


def kernel(inputs, table):
    return _gather_pe(inputs, table, _make_pos_enc())


# speedup vs baseline: 1.0902x; 1.0902x over previous
import functools

import jax
import jax.numpy as jnp
import numpy as np
from jax import lax
from jax.experimental import pallas as pl
from jax.experimental.pallas import tpu as pltpu
from jax.experimental.pallas import tpu_sc as plsc

N_TOKENS = 1000000
EMBED_DIM = 64
BATCH = 4096
SEQ = 200

_NC = 2
_NS = 16
_NW = _NC * _NS

_SEQ_PER_CHUNK = 2
_CHUNK = _SEQ_PER_CHUNK * SEQ
_PER_W = (BATCH // _NW) * SEQ
_NCHUNK = _PER_W // _CHUNK
_LANES = 16
_DBLK = EMBED_DIM // _LANES


def _make_pos_enc():
    pos = np.arange(SEQ, dtype=np.float32)[:, None]
    i = np.arange(0, EMBED_DIM, 2, dtype=np.float32)[None, :]
    angle = pos / np.power(10000.0, i / float(EMBED_DIM))
    pe = np.zeros((SEQ, EMBED_DIM), dtype=np.float32)
    pe[:, 0::2] = np.sin(angle)
    pe[:, 1::2] = np.cos(angle)
    return jnp.asarray(np.tile(pe, (_SEQ_PER_CHUNK, 1)))


def _sc_kernel(table_hbm, idx_hbm, pe_hbm, out_hbm,
               idx_v, rows_v, pe_v, gsem, ssem):
    wid = lax.axis_index("s") * _NC + lax.axis_index("c")
    base = wid * _PER_W
    pltpu.sync_copy(pe_hbm, pe_v)

    def start_gather(c, b):
        off = base + c * _CHUNK
        pltpu.sync_copy(idx_hbm.at[pl.ds(off, _CHUNK)], idx_v.at[b])
        pltpu.async_copy(table_hbm.at[idx_v.at[b]], rows_v.at[b], gsem.at[b])

    def wait_gather(b):
        pltpu.make_async_copy(
            table_hbm.at[idx_v.at[b]], rows_v.at[b], gsem.at[b]).wait()

    def start_store(c, b):
        off = base + c * _CHUNK
        pltpu.async_copy(rows_v.at[b], out_hbm.at[pl.ds(off, _CHUNK)],
                         ssem.at[b])

    def wait_store(c, b):
        off = base + c * _CHUNK
        pltpu.make_async_copy(
            rows_v.at[b], out_hbm.at[pl.ds(off, _CHUNK)], ssem.at[b]).wait()

    start_gather(0, 0)

    def chunk_body(c, carry):
        b = lax.rem(c, 2)
        b1 = 1 - b

        @pl.when(c + 1 < _NCHUNK)
        def _():
            @pl.when(c >= 1)
            def _():
                wait_store(c - 1, b1)
            start_gather(c + 1, b1)

        wait_gather(b)

        def add_body(s, carry2):
            for d in range(_DBLK):
                rows_v[b, s, pl.ds(d * _LANES, _LANES)] = (
                    rows_v[b, s, pl.ds(d * _LANES, _LANES)]
                    + pe_v[s, pl.ds(d * _LANES, _LANES)]
                )
            return carry2

        lax.fori_loop(0, _CHUNK, add_body, 0, unroll=4)
        start_store(c, b)
        return carry

    lax.fori_loop(0, _NCHUNK, chunk_body, 0)
    wait_store(_NCHUNK - 1, (_NCHUNK - 1) % 2)


@jax.jit
def _gather_pe(inputs, table, pe):
    idx = inputs.reshape(-1)
    mesh = plsc.VectorSubcoreMesh(core_axis_name="c", subcore_axis_name="s")
    k = functools.partial(
        pl.kernel,
        mesh=mesh,
        out_type=jax.ShapeDtypeStruct((BATCH * SEQ, EMBED_DIM), jnp.float32),
        scratch_types=[
            pltpu.VMEM((2, _CHUNK), jnp.int32),
            pltpu.VMEM((2, _CHUNK, EMBED_DIM), jnp.float32),
            pltpu.VMEM((_CHUNK, EMBED_DIM), jnp.float32),
            pltpu.SemaphoreType.DMA((2,)),
            pltpu.SemaphoreType.DMA((2,)),
        ],
        compiler_params=pltpu.CompilerParams(use_tc_tiling_on_sc=False),
    )(_sc_kernel)
    out = k(table, idx, pe)
    return out.reshape(BATCH, SEQ, EMBED_DIM)


def kernel(inputs, table):
    return _gather_pe(inputs, table, _make_pos_enc())
